# Initial kernel scaffold; baseline (speedup 1.0000x reference)
import jax
import jax.numpy as jnp
from jax.experimental import pallas as pl


def kernel(features, coors, batch_size):
    raise NotImplementedError("write your pallas kernel here")


# speedup vs baseline: 1.8664x; 1.8664x over previous
import functools

import jax
import jax.numpy as jnp
from jax import lax
from jax.experimental import pallas as pl
from jax.experimental.pallas import tpu as pltpu
from jax.experimental.pallas import tpu_sc as plsc

N, C = 320000, 128
NSEG = 8
NC, NS, L = 2, 16, 16
NW = NC * NS
RPW = N // NW
CH = 400
NCH = RPW // CH


def _sc_segment_max(features, bids):
    mesh = plsc.VectorSubcoreMesh(core_axis_name="c", subcore_axis_name="s")

    @functools.partial(
        pl.kernel,
        mesh=mesh,
        out_type=jax.ShapeDtypeStruct((NW, NSEG * C), jnp.float32),
        compiler_params=pltpu.CompilerParams(needs_layout_passes=False),
        scratch_types=[
            pltpu.VMEM((CH, C), jnp.float32),
            pltpu.VMEM((CH,), jnp.int32),
            pltpu.VMEM((NSEG * C,), jnp.float32),
        ],
    )
    def k(feat_hbm, bid_hbm, out_hbm, fbuf, cbuf, acc):
        wid = lax.axis_index("s") * NC + lax.axis_index("c")
        neg = jnp.full((L,), -jnp.inf, dtype=jnp.float32)
        for j in range(NSEG * C // L):
            acc[pl.ds(j * L, L)] = neg
        iota = lax.iota(jnp.int32, L)

        def chunk_body(c, carry):
            base = wid * RPW + c * CH
            pltpu.sync_copy(feat_hbm.at[pl.ds(base, CH)], fbuf)
            pltpu.sync_copy(bid_hbm.at[pl.ds(base, CH)], cbuf)

            def row_body(r, carry2):
                rsplat = jnp.full((L,), r, dtype=jnp.int32)
                b = plsc.load_gather(cbuf, [rsplat])
                b = jnp.minimum(b, NSEG - 1)
                abase = b * C
                for j in range(C // L):
                    x = fbuf[r, pl.ds(j * L, L)]
                    aidx = abase + j * L + iota
                    cur = plsc.load_gather(acc, [aidx])
                    plsc.store_scatter(acc, [aidx], jnp.maximum(cur, x))
                return carry2

            lax.fori_loop(0, CH, row_body, 0)
            return carry

        lax.fori_loop(0, NCH, chunk_body, 0)
        pltpu.sync_copy(acc, out_hbm.at[wid])

    return k(features, bids)


def kernel(features, coors, batch_size):
    bids = coors[:, 0].astype(jnp.int32)
    partials = _sc_segment_max(features, bids)
    return jnp.max(partials.reshape(NW, NSEG, C), axis=0)


# speedup vs baseline: 2.1944x; 1.1758x over previous
import functools

import jax
import jax.numpy as jnp
from jax import lax
from jax.experimental import pallas as pl
from jax.experimental.pallas import tpu as pltpu
from jax.experimental.pallas import tpu_sc as plsc

N, C = 320000, 128
NSEG = 8
NC, NS, L = 2, 16, 16
NW = NC * NS
RPW = N // NW
CH = 200
NCH = RPW // CH
NJ = C // L


def _sc_segment_max(features, bids):
    mesh = plsc.VectorSubcoreMesh(core_axis_name="c", subcore_axis_name="s")

    @functools.partial(
        pl.kernel,
        mesh=mesh,
        out_type=jax.ShapeDtypeStruct((NW, NSEG * C), jnp.float32),
        compiler_params=pltpu.CompilerParams(needs_layout_passes=False),
        scratch_types=(
            [pltpu.VMEM((CH, C), jnp.float32) for _ in range(2)]
            + [pltpu.VMEM((CH,), jnp.int32) for _ in range(2)]
            + [pltpu.VMEM((NSEG * L,), jnp.float32) for _ in range(NJ)]
            + [pltpu.VMEM((NSEG * C,), jnp.float32)]
            + [pltpu.SemaphoreType.DMA for _ in range(4)]
        ),
    )
    def k(feat_hbm, bid_hbm, out_hbm, fb0, fb1, cb0, cb1,
          a0, a1, a2, a3, a4, a5, a6, a7, accfin, fs0, fs1, cs0, cs1):
        wid = lax.axis_index("s") * NC + lax.axis_index("c")
        fbufs, cbufs = (fb0, fb1), (cb0, cb1)
        fsems, csems = (fs0, fs1), (cs0, cs1)
        accs = (a0, a1, a2, a3, a4, a5, a6, a7)

        neg = jnp.full((L,), -jnp.inf, dtype=jnp.float32)
        for a in accs:
            for s in range(NSEG):
                a[pl.ds(s * L, L)] = neg
        iota = lax.iota(jnp.int32, L)

        def dma_start(slot, c):
            base = wid * RPW + c * CH
            pltpu.async_copy(feat_hbm.at[pl.ds(base, CH)], fbufs[slot], fsems[slot])
            pltpu.async_copy(bid_hbm.at[pl.ds(base, CH)], cbufs[slot], csems[slot])

        def dma_wait(slot, c):
            base = wid * RPW + c * CH
            pltpu.make_async_copy(
                feat_hbm.at[pl.ds(base, CH)], fbufs[slot], fsems[slot]).wait()
            pltpu.make_async_copy(
                bid_hbm.at[pl.ds(base, CH)], cbufs[slot], csems[slot]).wait()

        def process(slot):
            fb, cb = fbufs[slot], cbufs[slot]

            def row_body(r, carry2):
                rsplat = jnp.full((L,), r, dtype=jnp.int32)
                b = plsc.load_gather(cb, [rsplat])
                b = jnp.minimum(b, NSEG - 1)
                aidx = b * L + iota
                for j in range(NJ):
                    x = fb[r, pl.ds(j * L, L)]
                    cur = plsc.load_gather(accs[j], [aidx])
                    plsc.store_scatter(accs[j], [aidx], jnp.maximum(cur, x))
                return carry2

            lax.fori_loop(0, CH, row_body, 0, unroll=2)

        dma_start(0, 0)

        def outer(c2, carry):
            for s in range(2):
                c = 2 * c2 + s

                @pl.when(c + 1 < NCH)
                def _():
                    dma_start(1 - s, c + 1)

                dma_wait(s, c)
                process(s)
            return carry

        lax.fori_loop(0, NCH // 2, outer, 0)

        for s in range(NSEG):
            for j in range(NJ):
                accfin[pl.ds(s * C + j * L, L)] = accs[j][pl.ds(s * L, L)]
        pltpu.sync_copy(accfin, out_hbm.at[wid])

    return k(features, bids)


def kernel(features, coors, batch_size):
    bids = coors[:, 0].astype(jnp.int32)
    partials = _sc_segment_max(features, bids)
    return jnp.max(partials.reshape(NW, NSEG, C), axis=0)


# speedup vs baseline: 2.2454x; 1.0232x over previous
import functools

import jax
import jax.numpy as jnp
from jax import lax
from jax.experimental import pallas as pl
from jax.experimental.pallas import tpu as pltpu
from jax.experimental.pallas import tpu_sc as plsc

N, C = 320000, 128
NSEG = 8
NC, NS, L = 2, 16, 16
NW = NC * NS
RPW = N // NW
CH = 200
NCH = RPW // CH
NJ = C // L


def _sc_segment_max(features, bids):
    mesh = plsc.VectorSubcoreMesh(core_axis_name="c", subcore_axis_name="s")

    @functools.partial(
        pl.kernel,
        mesh=mesh,
        out_type=jax.ShapeDtypeStruct((NW, NSEG * C), jnp.float32),
        compiler_params=pltpu.CompilerParams(needs_layout_passes=False),
        scratch_types=(
            [pltpu.VMEM((CH, C), jnp.float32) for _ in range(2)]
            + [pltpu.VMEM((CH,), jnp.int32) for _ in range(2)]
            + [pltpu.VMEM((NSEG * L,), jnp.float32) for _ in range(NJ)]
            + [pltpu.VMEM((NSEG * C,), jnp.float32)]
            + [pltpu.SemaphoreType.DMA for _ in range(4)]
        ),
    )
    def k(feat_hbm, bid_hbm, out_hbm, fb0, fb1, cb0, cb1,
          a0, a1, a2, a3, a4, a5, a6, a7, accfin, fs0, fs1, cs0, cs1):
        wid = lax.axis_index("s") * NC + lax.axis_index("c")
        fbufs, cbufs = (fb0, fb1), (cb0, cb1)
        fsems, csems = (fs0, fs1), (cs0, cs1)
        accs = (a0, a1, a2, a3, a4, a5, a6, a7)

        neg = jnp.full((L,), -jnp.inf, dtype=jnp.float32)
        for a in accs:
            for s in range(NSEG):
                a[pl.ds(s * L, L)] = neg
        iota = lax.iota(jnp.int32, L)
        jcols = [j * L + iota for j in range(NJ)]

        def dma_start(slot, c):
            base = wid * RPW + c * CH
            pltpu.async_copy(feat_hbm.at[pl.ds(base, CH)], fbufs[slot], fsems[slot])
            pltpu.async_copy(bid_hbm.at[pl.ds(base, CH)], cbufs[slot], csems[slot])

        def dma_wait(slot, c):
            base = wid * RPW + c * CH
            pltpu.make_async_copy(
                feat_hbm.at[pl.ds(base, CH)], fbufs[slot], fsems[slot]).wait()
            pltpu.make_async_copy(
                bid_hbm.at[pl.ds(base, CH)], cbufs[slot], csems[slot]).wait()

        def process(slot):
            fb, cb = fbufs[slot], cbufs[slot]

            def row_body(r, carry2):
                rsplat = jnp.full((L,), r, dtype=jnp.int32)
                b = plsc.load_gather(cb, [rsplat])
                b = jnp.minimum(b, NSEG - 1)
                aidx = b * L + iota
                for j in range(NJ):
                    x = plsc.load_gather(fb, [rsplat, jcols[j]])
                    cur = plsc.load_gather(accs[j], [aidx])
                    plsc.store_scatter(accs[j], [aidx], jnp.maximum(cur, x))
                return carry2

            lax.fori_loop(0, CH, row_body, 0, unroll=4)

        dma_start(0, 0)

        def outer(c2, carry):
            for s in range(2):
                c = 2 * c2 + s

                @pl.when(c + 1 < NCH)
                def _():
                    dma_start(1 - s, c + 1)

                dma_wait(s, c)
                process(s)
            return carry

        lax.fori_loop(0, NCH // 2, outer, 0)

        for s in range(NSEG):
            for j in range(NJ):
                accfin[pl.ds(s * C + j * L, L)] = accs[j][pl.ds(s * L, L)]
        pltpu.sync_copy(accfin, out_hbm.at[wid])

    return k(features, bids)


def kernel(features, coors, batch_size):
    bids = coors[:, 0].astype(jnp.int32)
    partials = _sc_segment_max(features, bids)
    return jnp.max(partials.reshape(NW, NSEG, C), axis=0)


# speedup vs baseline: 5.0899x; 2.2669x over previous
import functools

import jax
import jax.numpy as jnp
from jax import lax
from jax.experimental import pallas as pl
from jax.experimental.pallas import tpu as pltpu
from jax.experimental.pallas import tpu_sc as plsc

N, C = 320000, 128
NSEG = 8
NC, NS, L = 2, 16, 16
NW = NC * NS
RPW = N // NW
CH = 200
NCH = RPW // CH
NJ = C // L


def _sc_segment_max(features, bids):
    mesh = plsc.VectorSubcoreMesh(core_axis_name="c", subcore_axis_name="s")

    @functools.partial(
        pl.kernel,
        mesh=mesh,
        out_type=jax.ShapeDtypeStruct((NW, NSEG * C), jnp.float32),
        compiler_params=pltpu.CompilerParams(needs_layout_passes=False),
        scratch_types=(
            [pltpu.VMEM((CH, C), jnp.float32) for _ in range(2)]
            + [pltpu.VMEM((CH,), jnp.int32) for _ in range(2)]
            + [pltpu.VMEM((NSEG * L,), jnp.float32) for _ in range(NJ)]
            + [pltpu.VMEM((NSEG * C,), jnp.float32)]
            + [pltpu.SemaphoreType.DMA for _ in range(4)]
        ),
    )
    def k(feat_hbm, bid_hbm, out_hbm, fb0, fb1, cb0, cb1,
          a0, a1, a2, a3, a4, a5, a6, a7, accfin, fs0, fs1, cs0, cs1):
        wid = lax.axis_index("s") * NC + lax.axis_index("c")
        fbufs, cbufs = (fb0, fb1), (cb0, cb1)
        fsems, csems = (fs0, fs1), (cs0, cs1)
        accs = (a0, a1, a2, a3, a4, a5, a6, a7)

        neg = jnp.full((L,), -jnp.inf, dtype=jnp.float32)
        for a in accs:
            for s in range(NSEG):
                a[pl.ds(s * L, L)] = neg
        iota = lax.iota(jnp.int32, L)
        jcols = [j * L + iota for j in range(NJ)]

        def dma_start(slot, c):
            base = wid * RPW + c * CH
            pltpu.async_copy(feat_hbm.at[pl.ds(base, CH)], fbufs[slot], fsems[slot])
            pltpu.async_copy(bid_hbm.at[pl.ds(base, CH)], cbufs[slot], csems[slot])

        def dma_wait(slot, c):
            base = wid * RPW + c * CH
            pltpu.make_async_copy(
                feat_hbm.at[pl.ds(base, CH)], fbufs[slot], fsems[slot]).wait()
            pltpu.make_async_copy(
                bid_hbm.at[pl.ds(base, CH)], cbufs[slot], csems[slot]).wait()

        def process(slot):
            fb, cb = fbufs[slot], cbufs[slot]

            def row_body(r, carry2):
                rsplat = jnp.full((L,), r, dtype=jnp.int32)
                b = plsc.load_gather(cb, [rsplat])
                b = jnp.minimum(b, NSEG - 1)
                aidx = b * L + iota
                xs = [plsc.load_gather(fb, [rsplat, jcols[j]]) for j in range(NJ)]
                curs = [plsc.load_gather(accs[j], [aidx]) for j in range(NJ)]
                for j in range(NJ):
                    plsc.store_scatter(accs[j], [aidx], jnp.maximum(curs[j], xs[j]))
                return carry2

            lax.fori_loop(0, CH, row_body, 0, unroll=2)

        dma_start(0, 0)

        def outer(c2, carry):
            for s in range(2):
                c = 2 * c2 + s

                @pl.when(c + 1 < NCH)
                def _():
                    dma_start(1 - s, c + 1)

                dma_wait(s, c)
                process(s)
            return carry

        lax.fori_loop(0, NCH // 2, outer, 0)

        for s in range(NSEG):
            for j in range(NJ):
                accfin[pl.ds(s * C + j * L, L)] = accs[j][pl.ds(s * L, L)]
        pltpu.sync_copy(accfin, out_hbm.at[wid])

    return k(features, bids)


def kernel(features, coors, batch_size):
    bids = coors[:, 0].astype(jnp.int32)
    partials = _sc_segment_max(features, bids)
    return jnp.max(partials.reshape(NW, NSEG, C), axis=0)


# speedup vs baseline: 6.1717x; 1.2125x over previous
import functools

import jax
import jax.numpy as jnp
from jax import lax
from jax.experimental import pallas as pl
from jax.experimental.pallas import tpu as pltpu
from jax.experimental.pallas import tpu_sc as plsc

N, C = 320000, 128
NSEG = 8
NC, NS, L = 2, 16, 16
NW = NC * NS
RPW = N // NW
CH = 400
NCH = RPW // CH
NG = CH // L
NJ = C // L


def _sc_segment_max(features, bids):
    mesh = plsc.VectorSubcoreMesh(core_axis_name="c", subcore_axis_name="s")

    @functools.partial(
        pl.kernel,
        mesh=mesh,
        out_type=jax.ShapeDtypeStruct((NW, NSEG * C), jnp.float32),
        compiler_params=pltpu.CompilerParams(needs_layout_passes=False),
        scratch_types=(
            [pltpu.VMEM((CH, C), jnp.float32) for _ in range(2)]
            + [pltpu.VMEM((CH,), jnp.int32) for _ in range(2)]
            + [pltpu.VMEM((NSEG * L,), jnp.float32) for _ in range(NJ)]
            + [pltpu.VMEM((NSEG * C,), jnp.float32)]
            + [pltpu.SemaphoreType.DMA for _ in range(4)]
        ),
    )
    def k(feat_hbm, bid_hbm, out_hbm, fb0, fb1, cb0, cb1,
          a0, a1, a2, a3, a4, a5, a6, a7, accfin, fs0, fs1, cs0, cs1):
        wid = lax.axis_index("s") * NC + lax.axis_index("c")
        fbufs, cbufs = (fb0, fb1), (cb0, cb1)
        fsems, csems = (fs0, fs1), (cs0, cs1)
        accs = (a0, a1, a2, a3, a4, a5, a6, a7)

        neg = jnp.full((L,), -jnp.inf, dtype=jnp.float32)
        for a in accs:
            for s in range(NSEG):
                a[pl.ds(s * L, L)] = neg
        iota = lax.iota(jnp.int32, L)
        jcols = [j * L + iota for j in range(NJ)]

        def dma_start(slot, c):
            base = wid * RPW + c * CH
            pltpu.async_copy(feat_hbm.at[pl.ds(base, CH)], fbufs[slot], fsems[slot])
            pltpu.async_copy(bid_hbm.at[pl.ds(base, CH)], cbufs[slot], csems[slot])

        def dma_wait(slot, c):
            base = wid * RPW + c * CH
            pltpu.make_async_copy(
                feat_hbm.at[pl.ds(base, CH)], fbufs[slot], fsems[slot]).wait()
            pltpu.make_async_copy(
                bid_hbm.at[pl.ds(base, CH)], cbufs[slot], csems[slot]).wait()

        def process(slot):
            fb, cb = fbufs[slot], cbufs[slot]

            def group_body(g, carry2):
                r0 = g * L
                bvec = cb[pl.ds(r0, L)]
                abase = jnp.minimum(bvec, NSEG - 1) * L

                def do_row(r, aidx):
                    xs = [plsc.load_gather(fb, [jnp.full((L,), r, jnp.int32),
                                                jcols[j]]) for j in range(NJ)]
                    curs = [plsc.load_gather(accs[j], [aidx]) for j in range(NJ)]
                    for j in range(NJ):
                        plsc.store_scatter(accs[j], [aidx],
                                           jnp.maximum(curs[j], xs[j]))

                for u in range(L):
                    do_row(r0 + u, jnp.full((L,), abase[u], jnp.int32) + iota)
                return carry2

            lax.fori_loop(0, NG, group_body, 0)

        dma_start(0, 0)

        def outer(c2, carry):
            for s in range(2):
                c = 2 * c2 + s

                @pl.when(c + 1 < NCH)
                def _():
                    dma_start(1 - s, c + 1)

                dma_wait(s, c)
                process(s)
            return carry

        lax.fori_loop(0, NCH // 2, outer, 0)
        if NCH % 2:
            dma_wait(0, NCH - 1)
            process(0)

        for s in range(NSEG):
            for j in range(NJ):
                accfin[pl.ds(s * C + j * L, L)] = accs[j][pl.ds(s * L, L)]
        pltpu.sync_copy(accfin, out_hbm.at[wid])

    return k(features, bids)


def kernel(features, coors, batch_size):
    bids = coors[:, 0].astype(jnp.int32)
    partials = _sc_segment_max(features, bids)
    return jnp.max(partials.reshape(NW, NSEG, C), axis=0)
